# Initial kernel scaffold; baseline (speedup 1.0000x reference)
import jax
import jax.numpy as jnp
from jax.experimental import pallas as pl


def kernel(indices, table):
    raise NotImplementedError("write your pallas kernel here")


# speedup vs baseline: 1.2092x; 1.2092x over previous
import functools

import jax
import jax.numpy as jnp
from jax import lax
from jax.experimental import pallas as pl
from jax.experimental.pallas import tpu as pltpu
from jax.experimental.pallas import tpu_sc as plsc

_B = 4096 * 50
_D = 256
_NC = 2
_NS = 16
_NW = _NC * _NS
_BPW = _B // _NW
_K = 128
_NCHUNK = _BPW // _K

_mesh = plsc.VectorSubcoreMesh(core_axis_name="c", subcore_axis_name="s")


@functools.partial(
    pl.kernel,
    mesh=_mesh,
    out_type=jax.ShapeDtypeStruct((_B, _D), jnp.float32),
    scratch_types=[
        pltpu.VMEM((_BPW,), jnp.int32),
        pltpu.VMEM((_K, _D), jnp.float32),
        pltpu.SemaphoreType.DMA,
    ],
)
def _gather_all(idx_hbm, table_hbm, out_hbm, idx_v, rows_v, gsem):
    wid = lax.axis_index("s") * _NC + lax.axis_index("c")
    base = wid * _BPW
    pltpu.sync_copy(idx_hbm.at[pl.ds(base, _BPW)], idx_v)

    def body(c, carry):
        cbase = pl.multiple_of(c * _K, _K)
        pltpu.async_copy(
            table_hbm.at[idx_v.at[pl.ds(cbase, _K)]], rows_v, gsem
        ).wait()
        pltpu.sync_copy(rows_v, out_hbm.at[pl.ds(base + cbase, _K)])
        return carry

    lax.fori_loop(0, _NCHUNK, body, 0)


def kernel(indices, table):
    idx_flat = indices.reshape(-1).astype(jnp.int32)
    out = _gather_all(idx_flat, table)
    return out.reshape(indices.shape[0], indices.shape[1], _D)


# speedup vs baseline: 1.2901x; 1.0669x over previous
import functools

import jax
import jax.numpy as jnp
from jax import lax
from jax.experimental import pallas as pl
from jax.experimental.pallas import tpu as pltpu
from jax.experimental.pallas import tpu_sc as plsc

_B = 4096 * 50
_D = 256
_NC = 2
_NS = 16
_NW = _NC * _NS
_BPW = _B // _NW
_K = 128
_NCHUNK = _BPW // _K

_mesh = plsc.VectorSubcoreMesh(core_axis_name="c", subcore_axis_name="s")


_NBUF = 3


@functools.partial(
    pl.kernel,
    mesh=_mesh,
    out_type=jax.ShapeDtypeStruct((_B, _D), jnp.float32),
    scratch_types=[
        pltpu.VMEM((_BPW,), jnp.int32),
        pltpu.VMEM((_NBUF, _K, _D), jnp.float32),
        pltpu.SemaphoreType.DMA,
        pltpu.SemaphoreType.DMA,
    ],
)
def _gather_all(idx_hbm, table_hbm, out_hbm, idx_v, rows_v, gsem, ssem):
    wid = lax.axis_index("s") * _NC + lax.axis_index("c")
    base = wid * _BPW
    pltpu.sync_copy(idx_hbm.at[pl.ds(base, _BPW)], idx_v)

    def gather_copy(c, b):
        cbase = pl.multiple_of(c * _K, _K)
        return pltpu.make_async_copy(
            table_hbm.at[idx_v.at[pl.ds(cbase, _K)]], rows_v.at[b], gsem
        )

    def store_copy(c, b):
        cbase = pl.multiple_of(c * _K, _K)
        return pltpu.make_async_copy(
            rows_v.at[b], out_hbm.at[pl.ds(base + cbase, _K)], ssem
        )

    gather_copy(0, 0).start()
    gather_copy(1, 1).start()

    def body(c, carry):
        b = lax.rem(c, _NBUF)
        gather_copy(c, b).wait()

        @pl.when(c >= 1)
        def _():
            store_copy(c - 1, lax.rem(c + 2, _NBUF)).wait()

        @pl.when(c + 2 < _NCHUNK)
        def _():
            gather_copy(c + 2, lax.rem(c + 2, _NBUF)).start()

        store_copy(c, b).start()
        return carry

    lax.fori_loop(0, _NCHUNK, body, 0)
    store_copy(_NCHUNK - 1, (_NCHUNK - 1) % _NBUF).wait()


def kernel(indices, table):
    idx_flat = indices.reshape(-1).astype(jnp.int32)
    out = _gather_all(idx_flat, table)
    return out.reshape(indices.shape[0], indices.shape[1], _D)
